# baseline (speedup 1.0000x reference)
import jax
import jax.numpy as jnp
from jax.experimental import pallas as pl
from jax.experimental.pallas import tpu as pltpu

_BM = 512


def _ggc_body(a_ref, ann_ref, h_ref, bias_ref, w_ref, u_ref, bin_ref,
              brec_ref, out_ref):
    c = h_ref.shape[-1]
    a = a_ref[0]
    ann = ann_ref[0]
    h = h_ref[0]
    x = jnp.dot(a, ann, preferred_element_type=jnp.float32) + bias_ref[0]
    xw = jnp.dot(x, w_ref[:], preferred_element_type=jnp.float32) + bin_ref[:]
    hu = jnp.dot(h, u_ref[:], preferred_element_type=jnp.float32) + brec_ref[:]
    z = jax.nn.sigmoid(xw[:, :c] + hu[:, :c])
    r = jax.nn.sigmoid(xw[:, c:2 * c] + hu[:, c:2 * c])
    hh = jnp.tanh(xw[:, 2 * c:] + r * hu[:, 2 * c:])
    out_ref[0] = z * h + (1.0 - z) * hh


@jax.jit
def kernel(adjacent, annotations, gc_bias, W, U, b_in, b_rec):
    b, n, c = annotations.shape
    bm = min(_BM, n)
    grid = (b, n // bm)
    out = pl.pallas_call(
        _ggc_body,
        grid=grid,
        in_specs=[
            pl.BlockSpec((1, bm, n), lambda i, j: (i, j, 0)),
            pl.BlockSpec((1, n, c), lambda i, j: (i, 0, 0)),
            pl.BlockSpec((1, bm, c), lambda i, j: (i, j, 0)),
            pl.BlockSpec((1, c), lambda i, j: (0, 0)),
            pl.BlockSpec((c, 3 * c), lambda i, j: (0, 0)),
            pl.BlockSpec((c, 3 * c), lambda i, j: (0, 0)),
            pl.BlockSpec((1, 3 * c), lambda i, j: (0, 0)),
            pl.BlockSpec((1, 3 * c), lambda i, j: (0, 0)),
        ],
        out_specs=pl.BlockSpec((1, bm, c), lambda i, j: (i, j, 0)),
        out_shape=jax.ShapeDtypeStruct((b, n, c), jnp.float32),
        compiler_params=pltpu.CompilerParams(
            dimension_semantics=("parallel", "arbitrary"),
        ),
    )(adjacent, annotations, annotations,
      gc_bias.reshape(1, c), W, U,
      b_in.reshape(1, 3 * c), b_rec.reshape(1, 3 * c))
    return out


# speedup vs baseline: 1.0133x; 1.0133x over previous
import jax
import jax.numpy as jnp
from jax.experimental import pallas as pl
from jax.experimental.pallas import tpu as pltpu

_BM = 1024


def _ggc_body(a_ref, ann_ref, h_ref, bias_ref, w_ref, u_ref, bin_ref,
              brec_ref, out_ref):
    c = h_ref.shape[-1]
    a = a_ref[0]
    ann = ann_ref[0]
    h = h_ref[0]
    x = jnp.dot(a, ann, preferred_element_type=jnp.float32) + bias_ref[0]
    xw = jnp.dot(x, w_ref[:], preferred_element_type=jnp.float32) + bin_ref[:]
    hu = jnp.dot(h, u_ref[:], preferred_element_type=jnp.float32) + brec_ref[:]
    z = jax.nn.sigmoid(xw[:, :c] + hu[:, :c])
    r = jax.nn.sigmoid(xw[:, c:2 * c] + hu[:, c:2 * c])
    hh = jnp.tanh(xw[:, 2 * c:] + r * hu[:, 2 * c:])
    out_ref[0] = z * h + (1.0 - z) * hh


@jax.jit
def kernel(adjacent, annotations, gc_bias, W, U, b_in, b_rec):
    b, n, c = annotations.shape
    bm = min(_BM, n)
    grid = (b, n // bm)
    out = pl.pallas_call(
        _ggc_body,
        grid=grid,
        in_specs=[
            pl.BlockSpec((1, bm, n), lambda i, j: (i, j, 0)),
            pl.BlockSpec((1, n, c), lambda i, j: (i, 0, 0)),
            pl.BlockSpec((1, bm, c), lambda i, j: (i, j, 0)),
            pl.BlockSpec((1, c), lambda i, j: (0, 0)),
            pl.BlockSpec((c, 3 * c), lambda i, j: (0, 0)),
            pl.BlockSpec((c, 3 * c), lambda i, j: (0, 0)),
            pl.BlockSpec((1, 3 * c), lambda i, j: (0, 0)),
            pl.BlockSpec((1, 3 * c), lambda i, j: (0, 0)),
        ],
        out_specs=pl.BlockSpec((1, bm, c), lambda i, j: (i, j, 0)),
        out_shape=jax.ShapeDtypeStruct((b, n, c), jnp.float32),
        compiler_params=pltpu.CompilerParams(
            dimension_semantics=("parallel", "arbitrary"),
        ),
    )(adjacent, annotations, annotations,
      gc_bias.reshape(1, c), W, U,
      b_in.reshape(1, 3 * c), b_rec.reshape(1, 3 * c))
    return out
